# Initial kernel scaffold; baseline (speedup 1.0000x reference)
import jax
import jax.numpy as jnp
from jax.experimental import pallas as pl


def kernel(x, edge_index, steps, W, a_src, a_dst):
    raise NotImplementedError("write your pallas kernel here")


# speedup vs baseline: 12.9329x; 12.9329x over previous
import jax
import jax.numpy as jnp
from jax import lax
from jax.experimental import pallas as pl
from jax.experimental.pallas import tpu as pltpu
from jax.experimental.pallas import tpu_sc as plsc

N = 10000
E = 320000
D = 128
NSTEP = 4
NC = 2
NS = 16
NW = NC * NS
CK = 80
EO = E // NW
NCH = EO // CK
NPAD = 10240
NROW = 624
F32 = jnp.float32
I32 = jnp.int32


def _tc_step_body(x_ref, o_ref, w_ref, as_ref, ad_ref,
                  xo_ref, h_ref, hs_ref, hd_ref):
    xt = x_ref[...] + o_ref[0] + o_ref[1]
    xo_ref[...] = xt
    h = jnp.dot(xt, w_ref[...], preferred_element_type=F32)
    h_ref[...] = h
    hs_ref[...] = jnp.sum(h * as_ref[...][None, :], axis=1)
    hd_ref[...] = jnp.sum(h * ad_ref[...][None, :], axis=1)


def _tc_step(x, o, w, a_src, a_dst):
    return pl.pallas_call(
        _tc_step_body,
        out_shape=[
            jax.ShapeDtypeStruct((N, D), F32),
            jax.ShapeDtypeStruct((N, D), F32),
            jax.ShapeDtypeStruct((N,), F32),
            jax.ShapeDtypeStruct((N,), F32),
        ],
    )(x, o, w, a_src, a_dst)


def _tc_add_body(x_ref, o_ref, xo_ref):
    xo_ref[...] = x_ref[...] + o_ref[0] + o_ref[1]


def _tc_add(x, o):
    return pl.pallas_call(
        _tc_add_body,
        out_shape=jax.ShapeDtypeStruct((N, D), F32),
    )(x, o)


def _sc_logits_body(asrc_hbm, adst_hbm, srcr_hbm, dstr_hbm,
                    e_hbm, dpart_hbm, gmaxs_hbm,
                    asrc_t, adst_t, e_sh, sidx, didx, zrow, maxb, maxall,
                    denom_acc, maxslab):
    c = lax.axis_index("c")
    s = lax.axis_index("s")
    w = c * NS + s
    gb = w * EO

    pltpu.sync_copy(asrc_hbm, asrc_t)
    pltpu.sync_copy(adst_hbm, adst_t)

    for k in range(8):
        zrow[pl.ds(k * 16, 16)] = jnp.zeros((16,), F32)
    for i in range(5):
        pltpu.sync_copy(zrow, denom_acc.at[pl.ds(s * 640 + i * 128, 128)])
    plsc.subcore_barrier()

    def e_body(i, mx):
        pltpu.sync_copy(srcr_hbm.at[w, i], sidx)
        pltpu.sync_copy(dstr_hbm.at[w, i], didx)
        for k in range(CK // 16):
            sv = sidx[0, pl.ds(k * 16, 16)]
            dv = didx[0, pl.ds(k * 16, 16)]
            ev = plsc.load_gather(asrc_t, [sv]) + plsc.load_gather(adst_t, [dv])
            ev = jnp.where(ev >= 0.0, ev, 0.2 * ev)
            e_sh[pl.ds(i * CK + k * 16, 16)] = ev
            mx = jnp.maximum(mx, ev)
        return mx
    mx = lax.fori_loop(0, NCH, e_body, jnp.full((16,), -jnp.inf, F32))
    pltpu.sync_copy(e_sh, e_hbm.at[pl.ds(gb, EO)])

    maxb[...] = jnp.full((16,), jnp.max(mx), F32)
    pltpu.sync_copy(maxb, maxslab.at[s])
    plsc.subcore_barrier()
    pltpu.sync_copy(maxslab, maxall)
    gv = maxall[0, :]
    for r in range(1, NS):
        gv = jnp.maximum(gv, maxall[r, :])
    g = jnp.full((16,), jnp.max(gv), F32)

    def x_body(i, _):
        pltpu.sync_copy(dstr_hbm.at[w, i], didx)
        for k in range(CK // 16):
            off = i * CK + k * 16
            e_sh[pl.ds(off, 16)] = jnp.exp(e_sh[pl.ds(off, 16)] - g)
        pltpu.sync_copy(e_sh.at[pl.ds(i * CK, CK)],
                        denom_acc.at[didx.at[0]], add=True)
        return 0
    lax.fori_loop(0, NCH, x_body, 0)
    plsc.subcore_barrier()

    pltpu.sync_copy(denom_acc.at[pl.ds(s * 640, 640)],
                    dpart_hbm.at[c, 0, pl.ds(s * 640, 640)])
    maxb[...] = g

    @pl.when(s == 0)
    def _pub_max():
        pltpu.sync_copy(maxb, gmaxs_hbm.at[c, 0])


_sc_logits = pl.kernel(
    _sc_logits_body,
    out_type=[
        jax.ShapeDtypeStruct((E,), F32),
        jax.ShapeDtypeStruct((NC, 1, NPAD), F32),
        jax.ShapeDtypeStruct((NC, 1, 16), F32),
    ],
    mesh=plsc.VectorSubcoreMesh(core_axis_name="c", subcore_axis_name="s"),
    scratch_types=[
        pltpu.VMEM((N,), F32),
        pltpu.VMEM((N,), F32),
        pltpu.VMEM((EO,), F32),
        pltpu.VMEM((1, CK), I32),
        pltpu.VMEM((1, CK), I32),
        pltpu.VMEM((128,), F32),
        pltpu.VMEM((16,), F32),
        pltpu.VMEM((NS, 16), F32),
        pltpu.VMEM_SHARED((NPAD,), F32),
        pltpu.VMEM_SHARED((NS, 16), F32),
    ],
    compiler_params=pltpu.CompilerParams(needs_layout_passes=False),
)


def _sc_msg_body(h_hbm, e_hbm, dpart_hbm, gmaxs_hbm, srcr_hbm, dstr_hbm,
                 al_hbm, outp_hbm,
                 e_my, denom_t, rows_v, zbuf, sidx, didx, d1c, g0b, g1b,
                 sem, out_acc):
    c = lax.axis_index("c")
    s = lax.axis_index("s")
    w = c * NS + s
    gb = w * EO

    zv = jnp.zeros((16,), F32)

    def zb_body(i, _):
        for k in range(D // 16):
            zbuf[i, pl.ds(k * 16, 16)] = zv
        return 0
    lax.fori_loop(0, 16, zb_body, 0)

    def zo_body(i, _):
        pltpu.sync_copy(zbuf, out_acc.at[pl.ds(s * NROW + i * 16, 16)])
        return 0
    lax.fori_loop(0, 40, zo_body, 0)

    pltpu.sync_copy(gmaxs_hbm.at[0, 0], g0b)
    pltpu.sync_copy(gmaxs_hbm.at[1, 0], g1b)
    g0 = g0b[...]
    g1 = g1b[...]
    gx = jnp.maximum(g0, g1)
    f0 = jnp.exp(g0 - gx)
    f1 = jnp.exp(g1 - gx)

    pltpu.sync_copy(dpart_hbm.at[0, 0], denom_t)

    def dc_body(i, _):
        pltpu.sync_copy(dpart_hbm.at[1, 0, pl.ds(i * 128, 128)], d1c)
        for k in range(8):
            sl = pl.ds(i * 128 + k * 16, 16)
            denom_t[sl] = denom_t[sl] * f0 + d1c[pl.ds(k * 16, 16)] * f1
        return 0
    lax.fori_loop(0, NPAD // 128, dc_body, 0)

    pltpu.sync_copy(e_hbm.at[pl.ds(gb, EO)], e_my)
    plsc.subcore_barrier()

    def m_body(i, _):
        pltpu.sync_copy(srcr_hbm.at[w, i], sidx)
        pltpu.sync_copy(dstr_hbm.at[w, i], didx)
        cp = pltpu.async_copy(h_hbm.at[sidx.at[0]], rows_v, sem)
        for k in range(CK // 16):
            off = i * CK + k * 16
            ex = jnp.exp(e_my[pl.ds(off, 16)] - gx)
            dv = didx[0, pl.ds(k * 16, 16)]
            dn = plsc.load_gather(denom_t, [dv])
            e_my[pl.ds(off, 16)] = ex / (dn + 1e-16)
        cp.wait()

        def scale(r):
            av16 = e_my[pl.ds(i * CK + r, 16)]
            for rr in range(16):
                av = jnp.full((16,), av16[rr], F32)
                for k in range(D // 16):
                    sl = pl.ds(k * 16, 16)
                    rows_v[r + rr, sl] = rows_v[r + rr, sl] * av
        plsc.parallel_loop(0, CK, step=16)(scale)
        pltpu.sync_copy(rows_v, out_acc.at[didx.at[0]], add=True)
        return 0
    lax.fori_loop(0, NCH, m_body, 0)

    pltpu.sync_copy(e_my, al_hbm.at[pl.ds(gb, EO)])
    plsc.subcore_barrier()

    for i in range(7):
        pltpu.sync_copy(out_acc.at[pl.ds(s * NROW + i * CK, CK)],
                        outp_hbm.at[c, pl.ds(s * NROW + i * CK, CK)])
    pltpu.sync_copy(out_acc.at[pl.ds(s * NROW + 7 * CK, 64)],
                    outp_hbm.at[c, pl.ds(s * NROW + 7 * CK, 64)])

    @pl.when(s == NS - 1)
    def _tail_rows():
        pltpu.sync_copy(out_acc.at[pl.ds(NS * NROW, N - NS * NROW)],
                        outp_hbm.at[c, pl.ds(NS * NROW, N - NS * NROW)])


_sc_msg = pl.kernel(
    _sc_msg_body,
    out_type=[
        jax.ShapeDtypeStruct((E,), F32),
        jax.ShapeDtypeStruct((NC, N, D), F32),
    ],
    mesh=plsc.VectorSubcoreMesh(core_axis_name="c", subcore_axis_name="s"),
    scratch_types=[
        pltpu.VMEM((EO,), F32),
        pltpu.VMEM((NPAD,), F32),
        pltpu.VMEM((CK, D), F32),
        pltpu.VMEM((16, D), F32),
        pltpu.VMEM((1, CK), I32),
        pltpu.VMEM((1, CK), I32),
        pltpu.VMEM((128,), F32),
        pltpu.VMEM((16,), F32),
        pltpu.VMEM((16,), F32),
        pltpu.SemaphoreType.DMA,
        pltpu.VMEM_SHARED((N, D), F32),
    ],
    compiler_params=pltpu.CompilerParams(needs_layout_passes=False),
)


def kernel(x, edge_index, steps, W, a_src, a_dst):
    x0 = x + (jnp.asarray(steps) - NSTEP).astype(x.dtype)
    srcr = edge_index[0].astype(I32).reshape(NW, NCH, 1, CK)
    dstr = edge_index[1].astype(I32).reshape(NW, NCH, 1, CK)

    o = jnp.zeros((NC, N, D), F32)
    xcur = x0
    xs, attn, rel = [], [], []
    for t in range(NSTEP):
        xcur, h, hs, hd = _tc_step(xcur, o, W, a_src, a_dst)
        if t > 0:
            xs.append(xcur)
        e, dpart, gmaxs = _sc_logits(hs, hd, srcr, dstr)
        al, o = _sc_msg(h, e, dpart, gmaxs, srcr, dstr)
        attn.append(al)
        rel.append(e)
    xs.append(_tc_add(xcur, o))
    x_s = jnp.stack(xs, axis=0)
    return (x_s, tuple(attn), tuple(rel))


# speedup vs baseline: 15.3900x; 1.1900x over previous
import jax
import jax.numpy as jnp
from jax import lax
from jax.experimental import pallas as pl
from jax.experimental.pallas import tpu as pltpu
from jax.experimental.pallas import tpu_sc as plsc

N = 10000
E = 320000
D = 128
NSTEP = 4
NC = 2
NS = 16
NW = NC * NS
CK = 80
EO = E // NW
NCH = EO // CK
NPAD = 10240
NROW = 624
F32 = jnp.float32
I32 = jnp.int32


def _tc_step_body(x_ref, o_ref, w_ref, as_ref, ad_ref,
                  xo_ref, h_ref, hs_ref, hd_ref):
    xt = x_ref[...] + o_ref[0] + o_ref[1]
    xo_ref[...] = xt
    h = jnp.dot(xt, w_ref[...], preferred_element_type=F32)
    h_ref[...] = h
    hs_ref[...] = jnp.sum(h * as_ref[...][None, :], axis=1)
    hd_ref[...] = jnp.sum(h * ad_ref[...][None, :], axis=1)


def _tc_step(x, o, w, a_src, a_dst):
    return pl.pallas_call(
        _tc_step_body,
        out_shape=[
            jax.ShapeDtypeStruct((N, D), F32),
            jax.ShapeDtypeStruct((N, D), F32),
            jax.ShapeDtypeStruct((N,), F32),
            jax.ShapeDtypeStruct((N,), F32),
        ],
    )(x, o, w, a_src, a_dst)


def _tc_add_body(x_ref, o_ref, xo_ref):
    xo_ref[...] = x_ref[...] + o_ref[0] + o_ref[1]


def _tc_add(x, o):
    return pl.pallas_call(
        _tc_add_body,
        out_shape=jax.ShapeDtypeStruct((N, D), F32),
    )(x, o)


def _sc_logits_body(asrc_hbm, adst_hbm, srcr_hbm, dstr_hbm,
                    e_hbm, dpart_hbm, gmaxs_hbm,
                    asrc_t, adst_t, e_sh, sidx, didx, zrow, maxb, maxall,
                    denom_acc, maxslab):
    c = lax.axis_index("c")
    s = lax.axis_index("s")
    w = c * NS + s
    gb = w * EO

    pltpu.sync_copy(asrc_hbm, asrc_t)
    pltpu.sync_copy(adst_hbm, adst_t)

    for k in range(8):
        zrow[pl.ds(k * 16, 16)] = jnp.zeros((16,), F32)
    for i in range(5):
        pltpu.sync_copy(zrow, denom_acc.at[pl.ds(s * 640 + i * 128, 128)])
    plsc.subcore_barrier()

    def e_body(i, mx):
        pltpu.sync_copy(srcr_hbm.at[w, i], sidx)
        pltpu.sync_copy(dstr_hbm.at[w, i], didx)
        for k in range(CK // 16):
            sv = sidx[0, pl.ds(k * 16, 16)]
            dv = didx[0, pl.ds(k * 16, 16)]
            ev = plsc.load_gather(asrc_t, [sv]) + plsc.load_gather(adst_t, [dv])
            ev = jnp.where(ev >= 0.0, ev, 0.2 * ev)
            e_sh[pl.ds(i * CK + k * 16, 16)] = ev
            mx = jnp.maximum(mx, ev)
        return mx
    mx = lax.fori_loop(0, NCH, e_body, jnp.full((16,), -jnp.inf, F32))
    pltpu.sync_copy(e_sh, e_hbm.at[pl.ds(gb, EO)])

    maxb[...] = jnp.full((16,), jnp.max(mx), F32)
    pltpu.sync_copy(maxb, maxslab.at[s])
    plsc.subcore_barrier()
    pltpu.sync_copy(maxslab, maxall)
    gv = maxall[0, :]
    for r in range(1, NS):
        gv = jnp.maximum(gv, maxall[r, :])
    g = jnp.full((16,), jnp.max(gv), F32)

    def x_body(i, _):
        pltpu.sync_copy(dstr_hbm.at[w, i], didx)
        for k in range(CK // 16):
            off = i * CK + k * 16
            e_sh[pl.ds(off, 16)] = jnp.exp(e_sh[pl.ds(off, 16)] - g)
        pltpu.sync_copy(e_sh.at[pl.ds(i * CK, CK)],
                        denom_acc.at[didx.at[0]], add=True)
        return 0
    lax.fori_loop(0, NCH, x_body, 0)
    plsc.subcore_barrier()

    pltpu.sync_copy(denom_acc.at[pl.ds(s * 640, 640)],
                    dpart_hbm.at[c, 0, pl.ds(s * 640, 640)])
    maxb[...] = g

    @pl.when(s == 0)
    def _pub_max():
        pltpu.sync_copy(maxb, gmaxs_hbm.at[c, 0])


_sc_logits = pl.kernel(
    _sc_logits_body,
    out_type=[
        jax.ShapeDtypeStruct((E,), F32),
        jax.ShapeDtypeStruct((NC, 1, NPAD), F32),
        jax.ShapeDtypeStruct((NC, 1, 16), F32),
    ],
    mesh=plsc.VectorSubcoreMesh(core_axis_name="c", subcore_axis_name="s"),
    scratch_types=[
        pltpu.VMEM((N,), F32),
        pltpu.VMEM((N,), F32),
        pltpu.VMEM((EO,), F32),
        pltpu.VMEM((1, CK), I32),
        pltpu.VMEM((1, CK), I32),
        pltpu.VMEM((128,), F32),
        pltpu.VMEM((16,), F32),
        pltpu.VMEM((NS, 16), F32),
        pltpu.VMEM_SHARED((NPAD,), F32),
        pltpu.VMEM_SHARED((NS, 16), F32),
    ],
    compiler_params=pltpu.CompilerParams(needs_layout_passes=False),
)


def _sc_msg_body(h_hbm, e_hbm, dpart_hbm, gmaxs_hbm, ei_hbm,
                 al_hbm, outp_hbm,
                 e_my, denom_t, rv0, rv1, zbuf, eidx0, eidx1, d1c, g0b, g1b,
                 gsem, ssem, out_acc):
    c = lax.axis_index("c")
    s = lax.axis_index("s")
    w = c * NS + s
    gb = w * EO

    zv = jnp.zeros((16,), F32)

    def zb_body(i, _):
        for k in range(D // 16):
            zbuf[i, pl.ds(k * 16, 16)] = zv
        return 0
    lax.fori_loop(0, 16, zb_body, 0)

    def zo_body(i, _):
        pltpu.sync_copy(zbuf, out_acc.at[pl.ds(s * NROW + i * 16, 16)])
        return 0
    lax.fori_loop(0, 40, zo_body, 0)

    pltpu.sync_copy(gmaxs_hbm.at[0, 0], g0b)
    pltpu.sync_copy(gmaxs_hbm.at[1, 0], g1b)
    g0 = g0b[...]
    g1 = g1b[...]
    gx = jnp.maximum(g0, g1)
    f0 = jnp.exp(g0 - gx)
    f1 = jnp.exp(g1 - gx)

    pltpu.sync_copy(dpart_hbm.at[0, 0], denom_t)

    def dc_body(i, _):
        pltpu.sync_copy(dpart_hbm.at[1, 0, pl.ds(i * 128, 128)], d1c)
        for k in range(8):
            sl = pl.ds(i * 128 + k * 16, 16)
            denom_t[sl] = denom_t[sl] * f0 + d1c[pl.ds(k * 16, 16)] * f1
        return 0
    lax.fori_loop(0, NPAD // 128, dc_body, 0)

    pltpu.sync_copy(e_hbm.at[pl.ds(gb, EO)], e_my)

    def alpha_chunk(i, eref):
        for k in range(CK // 16):
            off = i * CK + k * 16
            ex = jnp.exp(e_my[pl.ds(off, 16)] - gx)
            dv = eref[1, pl.ds(k * 16, 16)]
            dn = plsc.load_gather(denom_t, [dv])
            e_my[pl.ds(off, 16)] = ex / (dn + 1e-16)

    def scale_chunk(i, rv):
        def scale(r):
            av16 = e_my[pl.ds(i * CK + r, 16)]
            for rr in range(16):
                av = jnp.full((16,), av16[rr], F32)
                for k in range(D // 16):
                    sl = pl.ds(k * 16, 16)
                    rv[r + rr, sl] = rv[r + rr, sl] * av
        plsc.parallel_loop(0, CK, step=16)(scale)

    pltpu.sync_copy(ei_hbm.at[w, 0], eidx0)
    pltpu.async_copy(h_hbm.at[eidx0.at[0]], rv0, gsem)
    alpha_chunk(0, eidx0)
    plsc.subcore_barrier()

    def chunk_step(i, ep, rp, eq, rq, first, last):
        pltpu.make_async_copy(h_hbm.at[ep.at[0]], rp, gsem).wait()
        scale_chunk(i, rp)
        if not first:
            pltpu.make_async_copy(rq, out_acc.at[eq.at[1]], ssem).wait()
        pltpu.async_copy(rp, out_acc.at[ep.at[1]], ssem, add=True)
        if not last:
            pltpu.sync_copy(ei_hbm.at[w, i + 1], eq)
            pltpu.async_copy(h_hbm.at[eq.at[0]], rq, gsem)
            alpha_chunk(i + 1, eq)

    def pair(j, _):
        a = 2 * j
        pltpu.make_async_copy(h_hbm.at[eidx0.at[0]], rv0, gsem).wait()
        scale_chunk(a, rv0)

        @pl.when(j > 0)
        def _ws():
            pltpu.make_async_copy(rv1, out_acc.at[eidx1.at[1]], ssem).wait()
        pltpu.async_copy(rv0, out_acc.at[eidx0.at[1]], ssem, add=True)
        pltpu.sync_copy(ei_hbm.at[w, a + 1], eidx1)
        pltpu.async_copy(h_hbm.at[eidx1.at[0]], rv1, gsem)
        alpha_chunk(a + 1, eidx1)

        chunk_step(a + 1, eidx1, rv1, eidx0, rv0, False, False)
        return 0
    lax.fori_loop(0, (NCH - 1) // 2, pair, 0)
    chunk_step(NCH - 1, eidx0, rv0, eidx1, rv1, False, True)
    pltpu.make_async_copy(rv0, out_acc.at[eidx0.at[1]], ssem).wait()

    pltpu.sync_copy(e_my, al_hbm.at[pl.ds(gb, EO)])
    plsc.subcore_barrier()

    for i in range(7):
        pltpu.sync_copy(out_acc.at[pl.ds(s * NROW + i * CK, CK)],
                        outp_hbm.at[c, pl.ds(s * NROW + i * CK, CK)])
    pltpu.sync_copy(out_acc.at[pl.ds(s * NROW + 7 * CK, 64)],
                    outp_hbm.at[c, pl.ds(s * NROW + 7 * CK, 64)])

    @pl.when(s == NS - 1)
    def _tail_rows():
        pltpu.sync_copy(out_acc.at[pl.ds(NS * NROW, N - NS * NROW)],
                        outp_hbm.at[c, pl.ds(NS * NROW, N - NS * NROW)])


_sc_msg = pl.kernel(
    _sc_msg_body,
    out_type=[
        jax.ShapeDtypeStruct((E,), F32),
        jax.ShapeDtypeStruct((NC, N, D), F32),
    ],
    mesh=plsc.VectorSubcoreMesh(core_axis_name="c", subcore_axis_name="s"),
    scratch_types=[
        pltpu.VMEM((EO,), F32),
        pltpu.VMEM((NPAD,), F32),
        pltpu.VMEM((CK, D), F32),
        pltpu.VMEM((CK, D), F32),
        pltpu.VMEM((16, D), F32),
        pltpu.VMEM((2, CK), I32),
        pltpu.VMEM((2, CK), I32),
        pltpu.VMEM((128,), F32),
        pltpu.VMEM((16,), F32),
        pltpu.VMEM((16,), F32),
        pltpu.SemaphoreType.DMA,
        pltpu.SemaphoreType.DMA,
        pltpu.VMEM_SHARED((N, D), F32),
    ],
    compiler_params=pltpu.CompilerParams(needs_layout_passes=False),
)


def kernel(x, edge_index, steps, W, a_src, a_dst):
    x0 = x + (jnp.asarray(steps) - NSTEP).astype(x.dtype)
    srcr = edge_index[0].astype(I32).reshape(NW, NCH, 1, CK)
    dstr = edge_index[1].astype(I32).reshape(NW, NCH, 1, CK)
    ei = jnp.concatenate([srcr, dstr], axis=2)

    o = jnp.zeros((NC, N, D), F32)
    xcur = x0
    xs, attn, rel = [], [], []
    for t in range(NSTEP):
        xcur, h, hs, hd = _tc_step(xcur, o, W, a_src, a_dst)
        if t > 0:
            xs.append(xcur)
        e, dpart, gmaxs = _sc_logits(hs, hd, srcr, dstr)
        al, o = _sc_msg(h, e, dpart, gmaxs, ei)
        attn.append(al)
        rel.append(e)
    xs.append(_tc_add(xcur, o))
    x_s = jnp.stack(xs, axis=0)
    return (x_s, tuple(attn), tuple(rel))


# speedup vs baseline: 21.7293x; 1.4119x over previous
import jax
import jax.numpy as jnp
from jax import lax
from jax.experimental import pallas as pl
from jax.experimental.pallas import tpu as pltpu
from jax.experimental.pallas import tpu_sc as plsc

N = 10000
E = 320000
D = 128
NSTEP = 4
NC = 2
NS = 16
NW = NC * NS
CK = 80
EO = E // NW
NCH = EO // CK
NPAD = 10240
NROW = 624
F32 = jnp.float32
I32 = jnp.int32


def _tc_step_body(x_ref, o_ref, w_ref, as_ref, ad_ref,
                  xo_ref, h_ref, hs_ref, hd_ref):
    xt = x_ref[...] + o_ref[0] + o_ref[1]
    xo_ref[...] = xt
    h = jnp.dot(xt, w_ref[...], preferred_element_type=F32)
    h_ref[...] = h
    hs_ref[...] = jnp.sum(h * as_ref[...][None, :], axis=1)
    hd_ref[...] = jnp.sum(h * ad_ref[...][None, :], axis=1)


def _tc_step(x, o, w, a_src, a_dst):
    return pl.pallas_call(
        _tc_step_body,
        out_shape=[
            jax.ShapeDtypeStruct((N, D), F32),
            jax.ShapeDtypeStruct((N, D), F32),
            jax.ShapeDtypeStruct((N,), F32),
            jax.ShapeDtypeStruct((N,), F32),
        ],
    )(x, o, w, a_src, a_dst)


def _tc_add_body(x_ref, o_ref, xo_ref):
    xo_ref[...] = x_ref[...] + o_ref[0] + o_ref[1]


def _tc_add(x, o):
    return pl.pallas_call(
        _tc_add_body,
        out_shape=jax.ShapeDtypeStruct((N, D), F32),
    )(x, o)


NBC = 5
NBLK = NCH // NBC


def _sc_logits_body(asrc_hbm, adst_hbm, ei5_hbm,
                    e_hbm, dpart_hbm, gmaxs_hbm,
                    asrc_t, adst_t, e_sh, eb0, eb1, zrow, maxb, maxall,
                    isem, ssem, denom_acc, maxslab):
    c = lax.axis_index("c")
    s = lax.axis_index("s")
    w = c * NS + s
    gb = w * EO

    pltpu.sync_copy(asrc_hbm, asrc_t)
    pltpu.sync_copy(adst_hbm, adst_t)

    for k in range(8):
        zrow[pl.ds(k * 16, 16)] = jnp.zeros((16,), F32)
    for i in range(5):
        pltpu.sync_copy(zrow, denom_acc.at[pl.ds(s * 640 + i * 128, 128)])
    plsc.subcore_barrier()

    def wait_idx(buf):
        pltpu.make_async_copy(ei5_hbm.at[w, 0], buf, isem).wait()

    def e_block(blk, bp, bq, mx, first, last):
        if first:
            pltpu.sync_copy(ei5_hbm.at[w, blk], bp)
        else:
            wait_idx(bp)
        if not last:
            pltpu.async_copy(ei5_hbm.at[w, blk + 1], bq, isem)
        for t in range(NBC):
            for k in range(CK // 16):
                sv = bp[t, 0, pl.ds(k * 16, 16)]
                dv = bp[t, 1, pl.ds(k * 16, 16)]
                ev = (plsc.load_gather(asrc_t, [sv])
                      + plsc.load_gather(adst_t, [dv]))
                ev = jnp.where(ev >= 0.0, ev, 0.2 * ev)
                e_sh[pl.ds((blk * NBC + t) * CK + k * 16, 16)] = ev
                mx = jnp.maximum(mx, ev)
        return mx

    def e_pair(m, mx):
        a = 2 * m

        def blk_a_first():
            return e_block(a, eb0, eb1, mx, True, False)

        def blk_a_rest():
            return e_block(a, eb0, eb1, mx, False, False)
        mx2 = lax.cond(m == 0, blk_a_first, blk_a_rest)
        return e_block(a + 1, eb1, eb0, mx2, False, False)
    mx = lax.fori_loop(0, (NBLK - 1) // 2, e_pair,
                       jnp.full((16,), -jnp.inf, F32))
    mx = e_block(NBLK - 1, eb0, eb1, mx, False, True)
    pltpu.sync_copy(e_sh, e_hbm.at[pl.ds(gb, EO)])

    maxb[...] = jnp.full((16,), jnp.max(mx), F32)
    pltpu.sync_copy(maxb, maxslab.at[s])
    plsc.subcore_barrier()
    pltpu.sync_copy(maxslab, maxall)
    gv = maxall[0, :]
    for r in range(1, NS):
        gv = jnp.maximum(gv, maxall[r, :])
    g = jnp.full((16,), jnp.max(gv), F32)

    def drain_scat():
        for _ in range(NBC):
            pltpu.make_async_copy(e_sh.at[pl.ds(0, CK)],
                                  denom_acc.at[eb0.at[0, 1]], ssem).wait()

    def x_block(blk, bp, bq, first, last):
        if first:
            pltpu.sync_copy(ei5_hbm.at[w, blk], bp)
        else:
            wait_idx(bp)
            drain_scat()
        if not last:
            pltpu.async_copy(ei5_hbm.at[w, blk + 1], bq, isem)
        for t in range(NBC):
            ch = blk * NBC + t
            for k in range(CK // 16):
                off = ch * CK + k * 16
                e_sh[pl.ds(off, 16)] = jnp.exp(e_sh[pl.ds(off, 16)] - g)
            pltpu.async_copy(e_sh.at[pl.ds(ch * CK, CK)],
                             denom_acc.at[bp.at[t, 1]], ssem, add=True)

    def x_pair(m, _):
        a = 2 * m

        @pl.when(m == 0)
        def _first():
            x_block(a, eb0, eb1, True, False)

        @pl.when(m > 0)
        def _rest():
            x_block(a, eb0, eb1, False, False)
        x_block(a + 1, eb1, eb0, False, False)
        return 0
    lax.fori_loop(0, (NBLK - 1) // 2, x_pair, 0)
    x_block(NBLK - 1, eb0, eb1, False, True)
    drain_scat()
    plsc.subcore_barrier()

    pltpu.sync_copy(denom_acc.at[pl.ds(s * 640, 640)],
                    dpart_hbm.at[c, 0, pl.ds(s * 640, 640)])
    maxb[...] = g

    @pl.when(s == 0)
    def _pub_max():
        pltpu.sync_copy(maxb, gmaxs_hbm.at[c, 0])


_sc_logits = pl.kernel(
    _sc_logits_body,
    out_type=[
        jax.ShapeDtypeStruct((E,), F32),
        jax.ShapeDtypeStruct((NC, 1, NPAD), F32),
        jax.ShapeDtypeStruct((NC, 1, 16), F32),
    ],
    mesh=plsc.VectorSubcoreMesh(core_axis_name="c", subcore_axis_name="s"),
    scratch_types=[
        pltpu.VMEM((N,), F32),
        pltpu.VMEM((N,), F32),
        pltpu.VMEM((EO,), F32),
        pltpu.VMEM((NBC, 2, CK), I32),
        pltpu.VMEM((NBC, 2, CK), I32),
        pltpu.VMEM((128,), F32),
        pltpu.VMEM((16,), F32),
        pltpu.VMEM((NS, 16), F32),
        pltpu.SemaphoreType.DMA,
        pltpu.SemaphoreType.DMA,
        pltpu.VMEM_SHARED((NPAD,), F32),
        pltpu.VMEM_SHARED((NS, 16), F32),
    ],
    compiler_params=pltpu.CompilerParams(needs_layout_passes=False),
)


def _sc_msg_body(h_hbm, e_hbm, dpart_hbm, gmaxs_hbm, ei_hbm,
                 al_hbm, outp_hbm,
                 e_my, denom_t, rv0, rv1, zbuf, eidx0, eidx1, d1c, g0b, g1b,
                 gsem, ssem, out_acc):
    c = lax.axis_index("c")
    s = lax.axis_index("s")
    w = c * NS + s
    gb = w * EO

    zv = jnp.zeros((16,), F32)

    def zb_body(i, _):
        for k in range(D // 16):
            zbuf[i, pl.ds(k * 16, 16)] = zv
        return 0
    lax.fori_loop(0, 16, zb_body, 0)

    def zo_body(i, _):
        pltpu.sync_copy(zbuf, out_acc.at[pl.ds(s * NROW + i * 16, 16)])
        return 0
    lax.fori_loop(0, 40, zo_body, 0)

    pltpu.sync_copy(gmaxs_hbm.at[0, 0], g0b)
    pltpu.sync_copy(gmaxs_hbm.at[1, 0], g1b)
    g0 = g0b[...]
    g1 = g1b[...]
    gx = jnp.maximum(g0, g1)
    f0 = jnp.exp(g0 - gx)
    f1 = jnp.exp(g1 - gx)

    pltpu.sync_copy(dpart_hbm.at[0, 0], denom_t)

    def dc_body(i, _):
        pltpu.sync_copy(dpart_hbm.at[1, 0, pl.ds(i * 128, 128)], d1c)
        for k in range(8):
            sl = pl.ds(i * 128 + k * 16, 16)
            denom_t[sl] = denom_t[sl] * f0 + d1c[pl.ds(k * 16, 16)] * f1
        return 0
    lax.fori_loop(0, NPAD // 128, dc_body, 0)

    pltpu.sync_copy(e_hbm.at[pl.ds(gb, EO)], e_my)

    def alpha_chunk(i, eref):
        for k in range(CK // 16):
            off = i * CK + k * 16
            ex = jnp.exp(e_my[pl.ds(off, 16)] - gx)
            dv = eref[1, pl.ds(k * 16, 16)]
            dn = plsc.load_gather(denom_t, [dv])
            e_my[pl.ds(off, 16)] = ex / (dn + 1e-16)

    def scale_chunk(i, rv):
        def scale(r):
            av16 = e_my[pl.ds(i * CK + r, 16)]
            for rr in range(16):
                av = jnp.full((16,), av16[rr], F32)
                for k in range(D // 16):
                    sl = pl.ds(k * 16, 16)
                    rv[r + rr, sl] = rv[r + rr, sl] * av
        plsc.parallel_loop(0, CK, step=16)(scale)

    pltpu.sync_copy(ei_hbm.at[w, 0], eidx0)
    pltpu.async_copy(h_hbm.at[eidx0.at[0]], rv0, gsem)
    alpha_chunk(0, eidx0)
    plsc.subcore_barrier()

    def chunk_step(i, ep, rp, eq, rq, first, last):
        pltpu.make_async_copy(h_hbm.at[ep.at[0]], rp, gsem).wait()
        scale_chunk(i, rp)
        if not first:
            pltpu.make_async_copy(rq, out_acc.at[eq.at[1]], ssem).wait()
        pltpu.async_copy(rp, out_acc.at[ep.at[1]], ssem, add=True)
        if not last:
            pltpu.sync_copy(ei_hbm.at[w, i + 1], eq)
            pltpu.async_copy(h_hbm.at[eq.at[0]], rq, gsem)
            alpha_chunk(i + 1, eq)

    def pair(j, _):
        a = 2 * j
        pltpu.make_async_copy(h_hbm.at[eidx0.at[0]], rv0, gsem).wait()
        scale_chunk(a, rv0)

        @pl.when(j > 0)
        def _ws():
            pltpu.make_async_copy(rv1, out_acc.at[eidx1.at[1]], ssem).wait()
        pltpu.async_copy(rv0, out_acc.at[eidx0.at[1]], ssem, add=True)
        pltpu.sync_copy(ei_hbm.at[w, a + 1], eidx1)
        pltpu.async_copy(h_hbm.at[eidx1.at[0]], rv1, gsem)
        alpha_chunk(a + 1, eidx1)

        chunk_step(a + 1, eidx1, rv1, eidx0, rv0, False, False)
        return 0
    lax.fori_loop(0, (NCH - 1) // 2, pair, 0)
    chunk_step(NCH - 1, eidx0, rv0, eidx1, rv1, False, True)
    pltpu.make_async_copy(rv0, out_acc.at[eidx0.at[1]], ssem).wait()

    pltpu.sync_copy(e_my, al_hbm.at[pl.ds(gb, EO)])
    plsc.subcore_barrier()

    for i in range(7):
        pltpu.sync_copy(out_acc.at[pl.ds(s * NROW + i * CK, CK)],
                        outp_hbm.at[c, pl.ds(s * NROW + i * CK, CK)])
    pltpu.sync_copy(out_acc.at[pl.ds(s * NROW + 7 * CK, 64)],
                    outp_hbm.at[c, pl.ds(s * NROW + 7 * CK, 64)])

    @pl.when(s == NS - 1)
    def _tail_rows():
        pltpu.sync_copy(out_acc.at[pl.ds(NS * NROW, N - NS * NROW)],
                        outp_hbm.at[c, pl.ds(NS * NROW, N - NS * NROW)])


_sc_msg = pl.kernel(
    _sc_msg_body,
    out_type=[
        jax.ShapeDtypeStruct((E,), F32),
        jax.ShapeDtypeStruct((NC, N, D), F32),
    ],
    mesh=plsc.VectorSubcoreMesh(core_axis_name="c", subcore_axis_name="s"),
    scratch_types=[
        pltpu.VMEM((EO,), F32),
        pltpu.VMEM((NPAD,), F32),
        pltpu.VMEM((CK, D), F32),
        pltpu.VMEM((CK, D), F32),
        pltpu.VMEM((16, D), F32),
        pltpu.VMEM((2, CK), I32),
        pltpu.VMEM((2, CK), I32),
        pltpu.VMEM((128,), F32),
        pltpu.VMEM((16,), F32),
        pltpu.VMEM((16,), F32),
        pltpu.SemaphoreType.DMA,
        pltpu.SemaphoreType.DMA,
        pltpu.VMEM_SHARED((N, D), F32),
    ],
    compiler_params=pltpu.CompilerParams(needs_layout_passes=False),
)


def kernel(x, edge_index, steps, W, a_src, a_dst):
    x0 = x + (jnp.asarray(steps) - NSTEP).astype(x.dtype)
    srcr = edge_index[0].astype(I32).reshape(NW, NCH, 1, CK)
    dstr = edge_index[1].astype(I32).reshape(NW, NCH, 1, CK)
    ei = jnp.concatenate([srcr, dstr], axis=2)
    ei5 = ei.reshape(NW, NBLK, NBC, 2, CK)

    o = jnp.zeros((NC, N, D), F32)
    xcur = x0
    xs, attn, rel = [], [], []
    for t in range(NSTEP):
        xcur, h, hs, hd = _tc_step(xcur, o, W, a_src, a_dst)
        if t > 0:
            xs.append(xcur)
        e, dpart, gmaxs = _sc_logits(hs, hd, ei5)
        al, o = _sc_msg(h, e, dpart, gmaxs, ei)
        attn.append(al)
        rel.append(e)
    xs.append(_tc_add(xcur, o))
    x_s = jnp.stack(xs, axis=0)
    return (x_s, tuple(attn), tuple(rel))
